# Initial kernel scaffold; baseline (speedup 1.0000x reference)
import jax
import jax.numpy as jnp
from jax.experimental import pallas as pl


def kernel(x, batch, edge_index, edge_attr, node_mask, W_atom, b_atom, eps, Wb, bb, W1, b1, W2, b2, Wf1, bf1, Wf2, bf2):
    raise NotImplementedError("write your pallas kernel here")


# speedup vs baseline: 11.8468x; 11.8468x over previous
import functools

import jax
import jax.numpy as jnp
from jax import lax
from jax.experimental import pallas as pl
from jax.experimental.pallas import tpu as pltpu
from jax.experimental.pallas import tpu_sc as plsc

def _gelu(t):
  return 0.5 * t * (1.0 + lax.erf(t * 0.7071067811865476))


_NC = 2
_NS = 16
_NW = _NC * _NS

def _mesh():
  return plsc.VectorSubcoreMesh(
      core_axis_name="c", subcore_axis_name="s", num_cores=_NC,
      num_subcores=_NS)


def _sc_mask_gather(nm_pad, idx):
  (B,) = idx.shape
  T, Tm = nm_pad.shape
  assert Tm == 128
  bpw = B // _NW
  ch = 2000
  assert bpw % ch == 0 and ch % 16 == 0

  @functools.partial(
      pl.kernel,
      out_type=jax.ShapeDtypeStruct((B,), jnp.float32),
      mesh=_mesh(),
      compiler_params=pltpu.CompilerParams(needs_layout_passes=False),
      scratch_types=[
          pltpu.VMEM((T, Tm), jnp.float32),
          pltpu.VMEM((ch,), jnp.int32),
          pltpu.VMEM((ch,), jnp.float32),
      ],
  )
  def k(nm_hbm, idx_hbm, out_hbm, nm_v, idx_v, val_v):
    wid = lax.axis_index("s") * _NC + lax.axis_index("c")
    pltpu.sync_copy(nm_hbm, nm_v)
    base = wid * bpw

    def outer(j, _):
      off = base + j * ch
      pltpu.sync_copy(idx_hbm.at[pl.ds(off, ch)], idx_v)

      def inner(i, _):
        iv = idx_v[pl.ds(i * 16, 16)]
        hi = lax.shift_right_logical(iv, 7)
        lo = lax.bitwise_and(iv, 127)
        val_v[pl.ds(i * 16, 16)] = plsc.load_gather(nm_v, [hi, lo])
        return 0

      lax.fori_loop(0, ch // 16, inner, 0)
      pltpu.sync_copy(val_v, out_hbm.at[pl.ds(off, ch)])
      return 0

    lax.fori_loop(0, bpw // ch, outer, 0)

  return k(nm_pad, idx)


def _sc_row_gather(table, idx):
  B = idx.shape[0]
  D = table.shape[1]
  bpw = B // _NW
  ch = 80
  assert bpw % ch == 0

  @functools.partial(
      pl.kernel,
      out_type=jax.ShapeDtypeStruct((B, D), jnp.float32),
      mesh=_mesh(),
      scratch_types=[
          pltpu.VMEM((ch,), jnp.int32),
          pltpu.VMEM((ch, D), jnp.float32),
          pltpu.SemaphoreType.DMA,
      ],
  )
  def k(table_hbm, idx_hbm, out_hbm, idx_v, rows_v, sem):
    wid = lax.axis_index("s") * _NC + lax.axis_index("c")
    base = wid * bpw

    def body(j, _):
      off = base + j * ch
      pltpu.sync_copy(idx_hbm.at[pl.ds(off, ch)], idx_v)
      pltpu.async_copy(table_hbm.at[idx_v], rows_v, sem).wait()
      pltpu.sync_copy(rows_v, out_hbm.at[pl.ds(off, ch)])
      return 0

    lax.fori_loop(0, bpw // ch, body, 0)

  return k(table, idx)


def _sc_scatter_add(vals, dst, zeros_nd):
  B, D = vals.shape
  (E,) = dst.shape
  N = zeros_nd.shape[0]
  assert B == _NC * E
  epw = E // _NS
  ch = 80
  nr = N // _NS
  assert epw % ch == 0 and N % _NS == 0 and nr % ch == 0

  @functools.partial(
      pl.kernel,
      out_type=jax.ShapeDtypeStruct((_NC, N, D), jnp.float32),
      mesh=_mesh(),
      scratch_types=[
          pltpu.VMEM((ch,), jnp.int32),
          pltpu.VMEM((ch, D), jnp.float32),
          pltpu.VMEM_SHARED((N, D), jnp.float32),
      ],
  )
  def k(vals_hbm, dst_hbm, zeros_hbm, out_hbm, idx_v, buf_v, agg_s):
    c = lax.axis_index("c")
    s = lax.axis_index("s")
    pltpu.sync_copy(zeros_hbm.at[pl.ds(0, ch)], buf_v)

    def zbody(j, _):
      pltpu.sync_copy(buf_v, agg_s.at[pl.ds(s * nr + j * ch, ch)])
      return 0

    lax.fori_loop(0, nr // ch, zbody, 0)
    plsc.subcore_barrier()
    base_v = c * E + s * epw
    base_i = s * epw

    def body(j, _):
      pltpu.sync_copy(dst_hbm.at[pl.ds(base_i + j * ch, ch)], idx_v)
      pltpu.sync_copy(vals_hbm.at[pl.ds(base_v + j * ch, ch)], buf_v)
      pltpu.sync_copy(buf_v, agg_s.at[idx_v], add=True)
      return 0

    lax.fori_loop(0, epw // ch, body, 0)
    plsc.subcore_barrier()

    def obody(j, _):
      off = s * nr + j * ch
      pltpu.sync_copy(agg_s.at[pl.ds(off, ch)], buf_v)
      pltpu.sync_copy(buf_v, out_hbm.at[c].at[pl.ds(off, ch)])
      return 0

    lax.fori_loop(0, nr // ch, obody, 0)

  return k(vals, dst, zeros_nd)


def _tc_tile_encode(x, W, b, rc):
  n, _ = x.shape
  h = W.shape[1]
  br = 400
  nb = n // br

  def body(x_ref, w_ref, b_ref, o_ref):
    o_ref[...] = (
        jnp.dot(x_ref[...], w_ref[...], preferred_element_type=jnp.float32)
        + b_ref[...]
    )

  return pl.pallas_call(
      body,
      grid=(rc * nb,),
      in_specs=[
          pl.BlockSpec((br, x.shape[1]), lambda i: (i % nb, 0)),
          pl.BlockSpec(W.shape, lambda i: (0, 0)),
          pl.BlockSpec((1, h), lambda i: (0, 0)),
      ],
      out_specs=pl.BlockSpec((br, h), lambda i: (i, 0)),
      out_shape=jax.ShapeDtypeStruct((rc * n, h), jnp.float32),
  )(x, W, b.reshape(1, h))


def _tc_edge(gathered, edge_attr, Wb_l, bb_l, me_col):
  B, H = gathered.shape
  E, De = edge_attr.shape
  be = 800
  neb = E // be

  def body(g_ref, ea_ref, wb_ref, bb_ref, me_ref, o_ref):
    ee = (
        jnp.dot(ea_ref[...], wb_ref[...], preferred_element_type=jnp.float32)
        + bb_ref[...]
    )
    o_ref[...] = _gelu(g_ref[...] + ee) * me_ref[...]

  return pl.pallas_call(
      body,
      grid=(B // be,),
      in_specs=[
          pl.BlockSpec((be, H), lambda i: (i, 0)),
          pl.BlockSpec((be, De), lambda i: (i % neb, 0)),
          pl.BlockSpec(Wb_l.shape, lambda i: (0, 0)),
          pl.BlockSpec((1, H), lambda i: (0, 0)),
          pl.BlockSpec((be, 1), lambda i: (i, 0)),
      ],
      out_specs=pl.BlockSpec((be, H), lambda i: (i, 0)),
      out_shape=jax.ShapeDtypeStruct((B, H), jnp.float32),
  )(gathered, edge_attr, Wb_l, bb_l.reshape(1, H), me_col)


def _tc_node_update(hcur, agg2, nm_col, eps_l, W1l, b1l, W2l, b2l):
  B, H = hcur.shape
  br = 400

  def body(h_ref, a_ref, nm_ref, eps_ref, w1_ref, b1_ref, w2_ref, b2_ref, o_ref):
    h = h_ref[...]
    z = (1.0 + eps_ref[0, 0]) * h + nm_ref[...] * a_ref[...]
    t = _gelu(
        jnp.dot(z, w1_ref[...], preferred_element_type=jnp.float32) + b1_ref[...]
    )
    t = jnp.dot(t, w2_ref[...], preferred_element_type=jnp.float32) + b2_ref[...]
    o_ref[...] = h + _gelu(t)

  return pl.pallas_call(
      body,
      grid=(B // br,),
      in_specs=[
          pl.BlockSpec((br, H), lambda i: (i, 0)),
          pl.BlockSpec((br, H), lambda i: (i, 0)),
          pl.BlockSpec((br, 1), lambda i: (i, 0)),
          pl.BlockSpec((1, 1), lambda i: (0, 0)),
          pl.BlockSpec(W1l.shape, lambda i: (0, 0)),
          pl.BlockSpec((1, H), lambda i: (0, 0)),
          pl.BlockSpec(W2l.shape, lambda i: (0, 0)),
          pl.BlockSpec((1, H), lambda i: (0, 0)),
      ],
      out_specs=pl.BlockSpec((br, H), lambda i: (i, 0)),
      out_shape=jax.ShapeDtypeStruct((B, H), jnp.float32),
  )(hcur, agg2, nm_col, eps_l.reshape(1, 1), W1l, b1l.reshape(1, H), W2l,
    b2l.reshape(1, H))


def _tc_final(hcur, nm_col, Wf1, bf1, Wf2, bf2, rc):
  B, H = hcur.shape
  out_d = Wf2.shape[1]
  br = 400
  nb = B // br
  nbc = nb // rc

  def body(h_ref, nm_ref, w1_ref, b1_ref, w2_ref, b2_ref, o_ref, acc, den):
    i = pl.program_id(0)
    c = i // nbc

    @pl.when(i == 0)
    def _():
      acc[...] = jnp.zeros_like(acc)
      den[...] = jnp.zeros_like(den)

    y = jax.nn.relu(
        jnp.dot(h_ref[...], w1_ref[...], preferred_element_type=jnp.float32)
        + b1_ref[...]
    )
    y = jnp.dot(y, w2_ref[...], preferred_element_type=jnp.float32) + b2_ref[...]
    nm = nm_ref[...]
    acc[pl.ds(c, 1), :] += jnp.sum(y * nm, axis=0, keepdims=True)
    den[pl.ds(c, 1), :] += jnp.sum(nm, axis=0, keepdims=True)

    @pl.when(i == nb - 1)
    def _():
      o_ref[...] = acc[...] / den[...]

  return pl.pallas_call(
      body,
      grid=(nb,),
      in_specs=[
          pl.BlockSpec((br, H), lambda i: (i, 0)),
          pl.BlockSpec((br, 1), lambda i: (i, 0)),
          pl.BlockSpec(Wf1.shape, lambda i: (0, 0)),
          pl.BlockSpec((1, H), lambda i: (0, 0)),
          pl.BlockSpec(Wf2.shape, lambda i: (0, 0)),
          pl.BlockSpec((1, out_d), lambda i: (0, 0)),
      ],
      out_specs=pl.BlockSpec((rc, out_d), lambda i: (0, 0)),
      out_shape=jax.ShapeDtypeStruct((rc, out_d), jnp.float32),
      scratch_shapes=[
          pltpu.VMEM((rc, out_d), jnp.float32),
          pltpu.VMEM((rc, 1), jnp.float32),
      ],
  )(hcur, nm_col, Wf1, bf1.reshape(1, H), Wf2, bf2.reshape(1, out_d))


def kernel(x, batch, edge_index, edge_attr, node_mask, W_atom, b_atom, eps,
           Wb, bb, W1, b1, W2, b2, Wf1, bf1, Wf2, bf2):
  r, c, n, _ = node_mask.shape
  rc = r * c
  e = edge_index.shape[1]
  nl = Wb.shape[0]

  src = edge_index[0].astype(jnp.int32)
  dst = edge_index[1].astype(jnp.int32)
  offs = (jnp.arange(rc, dtype=jnp.int32) * n)[:, None]
  idx2 = (src[None, :] + offs).reshape(-1)

  nm_flat = node_mask.reshape(rc * n)
  nm_col = nm_flat.reshape(rc * n, 1)
  n2 = -(-n // (80 * _NS)) * (80 * _NS)
  zeros_nd = jnp.zeros((n2, W_atom.shape[1]), jnp.float32)

  trows = (rc * n + 127) // 128
  nm_pad = jnp.zeros((trows * 128,), jnp.float32).at[: rc * n].set(
      nm_flat).reshape(trows, 128)
  me_col = _sc_mask_gather(nm_pad, idx2).reshape(rc * e, 1)

  hcur = _tc_tile_encode(x, W_atom, b_atom, rc)
  for l in range(nl):
    gathered = _sc_row_gather(hcur, idx2)
    q = _tc_edge(gathered, edge_attr, Wb[l], bb[l], me_col)
    agg = _sc_scatter_add(q, dst, zeros_nd)[:, :n, :]
    hcur = _tc_node_update(hcur, agg.reshape(rc * n, -1), nm_col, eps[l],
                           W1[l], b1[l], W2[l], b2[l])

  y = _tc_final(hcur, nm_col, Wf1, bf1, Wf2, bf2, rc)
  return y.reshape(r, c, -1)


# speedup vs baseline: 14.2132x; 1.1998x over previous
import functools

import jax
import jax.numpy as jnp
from jax import lax
from jax.experimental import pallas as pl
from jax.experimental.pallas import tpu as pltpu
from jax.experimental.pallas import tpu_sc as plsc

def _gelu(t):
  return 0.5 * t * (1.0 + lax.erf(t * 0.7071067811865476))


_NC = 2
_NS = 16
_NW = _NC * _NS

def _mesh():
  return plsc.VectorSubcoreMesh(
      core_axis_name="c", subcore_axis_name="s", num_cores=_NC,
      num_subcores=_NS)


def _sc_mask_gather(nm_pad, idx):
  (B,) = idx.shape
  T, Tm = nm_pad.shape
  assert Tm == 128
  bpw = B // _NW
  ch = 2000
  assert bpw % ch == 0 and ch % 16 == 0

  @functools.partial(
      pl.kernel,
      out_type=jax.ShapeDtypeStruct((B,), jnp.float32),
      mesh=_mesh(),
      compiler_params=pltpu.CompilerParams(needs_layout_passes=False),
      scratch_types=[
          pltpu.VMEM((T, Tm), jnp.float32),
          pltpu.VMEM((ch,), jnp.int32),
          pltpu.VMEM((ch,), jnp.float32),
      ],
  )
  def k(nm_hbm, idx_hbm, out_hbm, nm_v, idx_v, val_v):
    wid = lax.axis_index("s") * _NC + lax.axis_index("c")
    pltpu.sync_copy(nm_hbm, nm_v)
    base = wid * bpw

    def outer(j, _):
      off = base + j * ch
      pltpu.sync_copy(idx_hbm.at[pl.ds(off, ch)], idx_v)

      def inner(i, _):
        iv = idx_v[pl.ds(i * 16, 16)]
        hi = lax.shift_right_logical(iv, 7)
        lo = lax.bitwise_and(iv, 127)
        val_v[pl.ds(i * 16, 16)] = plsc.load_gather(nm_v, [hi, lo])
        return 0

      lax.fori_loop(0, ch // 16, inner, 0)
      pltpu.sync_copy(val_v, out_hbm.at[pl.ds(off, ch)])
      return 0

    lax.fori_loop(0, bpw // ch, outer, 0)

  return k(nm_pad, idx)


def _sc_row_gather(table, idx):
  B = idx.shape[0]
  D = table.shape[1]
  bpw = B // _NW
  ch = 80
  nsub = 5
  big = ch * nsub
  assert bpw % big == 0

  @functools.partial(
      pl.kernel,
      out_type=jax.ShapeDtypeStruct((B, D), jnp.float32),
      mesh=_mesh(),
      scratch_types=[
          pltpu.VMEM((big,), jnp.int32),
          pltpu.VMEM((big, D), jnp.float32),
          pltpu.SemaphoreType.DMA,
      ],
  )
  def k(table_hbm, idx_hbm, out_hbm, idx_v, rows_v, sem):
    wid = lax.axis_index("s") * _NC + lax.axis_index("c")
    base = wid * bpw

    def body(j, _):
      off = base + j * big
      pltpu.sync_copy(idx_hbm.at[pl.ds(off, big)], idx_v)
      cps = [
          pltpu.async_copy(
              table_hbm.at[idx_v.at[pl.ds(kk * ch, ch)]],
              rows_v.at[pl.ds(kk * ch, ch)], sem)
          for kk in range(nsub)
      ]
      for cp in cps:
        cp.wait()
      pltpu.sync_copy(rows_v, out_hbm.at[pl.ds(off, big)])
      return 0

    lax.fori_loop(0, bpw // big, body, 0)

  return k(table, idx)


def _sc_scatter_add(vals, dst, zeros_nd):
  B, D = vals.shape
  (E,) = dst.shape
  N = zeros_nd.shape[0]
  assert B == _NC * E
  epw = E // _NS
  ch = 40
  nsub = 5
  big = ch * nsub
  nr = N // _NS
  assert epw % big == 0 and N % _NS == 0 and nr % ch == 0 and ch % 8 == 0
  dst3d = dst.reshape(E // big, nsub, ch)

  @functools.partial(
      pl.kernel,
      out_type=jax.ShapeDtypeStruct((_NC, N, D), jnp.float32),
      mesh=_mesh(),
      scratch_types=[
          pltpu.VMEM((nsub, ch), jnp.int32),
          pltpu.VMEM((big, D), jnp.float32),
          pltpu.VMEM_SHARED((N, D), jnp.float32),
      ],
  )
  def k(vals_hbm, dst_hbm, zeros_hbm, out_hbm, idx_v, buf_v, agg_s):
    c = lax.axis_index("c")
    s = lax.axis_index("s")
    pltpu.sync_copy(zeros_hbm.at[pl.ds(0, ch)], buf_v.at[pl.ds(0, ch)])

    def zbody(j, _):
      pltpu.sync_copy(buf_v.at[pl.ds(0, ch)],
                      agg_s.at[pl.ds(s * nr + j * ch, ch)])
      return 0

    lax.fori_loop(0, nr // ch, zbody, 0)
    plsc.subcore_barrier()
    base_v = c * E + s * epw
    base_r = (s * epw) // big

    def body(j, _):
      pltpu.sync_copy(dst_hbm.at[base_r + j], idx_v)
      pltpu.sync_copy(vals_hbm.at[pl.ds(base_v + j * big, big)], buf_v)
      for kk in range(nsub):
        pltpu.sync_copy(buf_v.at[pl.ds(kk * ch, ch)], agg_s.at[idx_v.at[kk]],
                        add=True)
      return 0

    lax.fori_loop(0, epw // big, body, 0)
    plsc.subcore_barrier()

    def obody(j, _):
      off = s * nr + j * ch
      pltpu.sync_copy(agg_s.at[pl.ds(off, ch)], buf_v.at[pl.ds(0, ch)])
      pltpu.sync_copy(buf_v.at[pl.ds(0, ch)], out_hbm.at[c].at[pl.ds(off, ch)])
      return 0

    lax.fori_loop(0, nr // ch, obody, 0)

  return k(vals, dst3d, zeros_nd)


def _tc_tile_encode(x, W, b, rc):
  n, _ = x.shape
  h = W.shape[1]
  br = 400
  nb = n // br

  def body(x_ref, w_ref, b_ref, o_ref):
    o_ref[...] = (
        jnp.dot(x_ref[...], w_ref[...], preferred_element_type=jnp.float32)
        + b_ref[...]
    )

  return pl.pallas_call(
      body,
      grid=(rc * nb,),
      in_specs=[
          pl.BlockSpec((br, x.shape[1]), lambda i: (i % nb, 0)),
          pl.BlockSpec(W.shape, lambda i: (0, 0)),
          pl.BlockSpec((1, h), lambda i: (0, 0)),
      ],
      out_specs=pl.BlockSpec((br, h), lambda i: (i, 0)),
      out_shape=jax.ShapeDtypeStruct((rc * n, h), jnp.float32),
  )(x, W, b.reshape(1, h))


def _tc_edge(gathered, edge_attr, Wb_l, bb_l, me_col):
  B, H = gathered.shape
  E, De = edge_attr.shape
  be = 800
  neb = E // be

  def body(g_ref, ea_ref, wb_ref, bb_ref, me_ref, o_ref):
    ee = (
        jnp.dot(ea_ref[...], wb_ref[...], preferred_element_type=jnp.float32)
        + bb_ref[...]
    )
    o_ref[...] = _gelu(g_ref[...] + ee) * me_ref[...]

  return pl.pallas_call(
      body,
      grid=(B // be,),
      in_specs=[
          pl.BlockSpec((be, H), lambda i: (i, 0)),
          pl.BlockSpec((be, De), lambda i: (i % neb, 0)),
          pl.BlockSpec(Wb_l.shape, lambda i: (0, 0)),
          pl.BlockSpec((1, H), lambda i: (0, 0)),
          pl.BlockSpec((be, 1), lambda i: (i, 0)),
      ],
      out_specs=pl.BlockSpec((be, H), lambda i: (i, 0)),
      out_shape=jax.ShapeDtypeStruct((B, H), jnp.float32),
  )(gathered, edge_attr, Wb_l, bb_l.reshape(1, H), me_col)


def _tc_node_update(hcur, agg2, nm_col, eps_l, W1l, b1l, W2l, b2l):
  B, H = hcur.shape
  br = 400

  def body(h_ref, a_ref, nm_ref, eps_ref, w1_ref, b1_ref, w2_ref, b2_ref, o_ref):
    h = h_ref[...]
    z = (1.0 + eps_ref[0, 0]) * h + nm_ref[...] * a_ref[...]
    t = _gelu(
        jnp.dot(z, w1_ref[...], preferred_element_type=jnp.float32) + b1_ref[...]
    )
    t = jnp.dot(t, w2_ref[...], preferred_element_type=jnp.float32) + b2_ref[...]
    o_ref[...] = h + _gelu(t)

  return pl.pallas_call(
      body,
      grid=(B // br,),
      in_specs=[
          pl.BlockSpec((br, H), lambda i: (i, 0)),
          pl.BlockSpec((br, H), lambda i: (i, 0)),
          pl.BlockSpec((br, 1), lambda i: (i, 0)),
          pl.BlockSpec((1, 1), lambda i: (0, 0)),
          pl.BlockSpec(W1l.shape, lambda i: (0, 0)),
          pl.BlockSpec((1, H), lambda i: (0, 0)),
          pl.BlockSpec(W2l.shape, lambda i: (0, 0)),
          pl.BlockSpec((1, H), lambda i: (0, 0)),
      ],
      out_specs=pl.BlockSpec((br, H), lambda i: (i, 0)),
      out_shape=jax.ShapeDtypeStruct((B, H), jnp.float32),
  )(hcur, agg2, nm_col, eps_l.reshape(1, 1), W1l, b1l.reshape(1, H), W2l,
    b2l.reshape(1, H))


def _tc_final(hcur, nm_col, Wf1, bf1, Wf2, bf2, rc):
  B, H = hcur.shape
  out_d = Wf2.shape[1]
  br = 400
  nb = B // br
  nbc = nb // rc

  def body(h_ref, nm_ref, w1_ref, b1_ref, w2_ref, b2_ref, o_ref, acc, den):
    i = pl.program_id(0)
    c = i // nbc

    @pl.when(i == 0)
    def _():
      acc[...] = jnp.zeros_like(acc)
      den[...] = jnp.zeros_like(den)

    y = jax.nn.relu(
        jnp.dot(h_ref[...], w1_ref[...], preferred_element_type=jnp.float32)
        + b1_ref[...]
    )
    y = jnp.dot(y, w2_ref[...], preferred_element_type=jnp.float32) + b2_ref[...]
    nm = nm_ref[...]
    acc[pl.ds(c, 1), :] += jnp.sum(y * nm, axis=0, keepdims=True)
    den[pl.ds(c, 1), :] += jnp.sum(nm, axis=0, keepdims=True)

    @pl.when(i == nb - 1)
    def _():
      o_ref[...] = acc[...] / den[...]

  return pl.pallas_call(
      body,
      grid=(nb,),
      in_specs=[
          pl.BlockSpec((br, H), lambda i: (i, 0)),
          pl.BlockSpec((br, 1), lambda i: (i, 0)),
          pl.BlockSpec(Wf1.shape, lambda i: (0, 0)),
          pl.BlockSpec((1, H), lambda i: (0, 0)),
          pl.BlockSpec(Wf2.shape, lambda i: (0, 0)),
          pl.BlockSpec((1, out_d), lambda i: (0, 0)),
      ],
      out_specs=pl.BlockSpec((rc, out_d), lambda i: (0, 0)),
      out_shape=jax.ShapeDtypeStruct((rc, out_d), jnp.float32),
      scratch_shapes=[
          pltpu.VMEM((rc, out_d), jnp.float32),
          pltpu.VMEM((rc, 1), jnp.float32),
      ],
  )(hcur, nm_col, Wf1, bf1.reshape(1, H), Wf2, bf2.reshape(1, out_d))


def kernel(x, batch, edge_index, edge_attr, node_mask, W_atom, b_atom, eps,
           Wb, bb, W1, b1, W2, b2, Wf1, bf1, Wf2, bf2):
  r, c, n, _ = node_mask.shape
  rc = r * c
  e = edge_index.shape[1]
  nl = Wb.shape[0]

  src = edge_index[0].astype(jnp.int32)
  dst = edge_index[1].astype(jnp.int32)
  offs = (jnp.arange(rc, dtype=jnp.int32) * n)[:, None]
  idx2 = (src[None, :] + offs).reshape(-1)

  nm_flat = node_mask.reshape(rc * n)
  nm_col = nm_flat.reshape(rc * n, 1)
  n2 = -(-n // (80 * _NS)) * (80 * _NS)
  zeros_nd = jnp.zeros((n2, W_atom.shape[1]), jnp.float32)

  trows = (rc * n + 127) // 128
  nm_pad = jnp.zeros((trows * 128,), jnp.float32).at[: rc * n].set(
      nm_flat).reshape(trows, 128)
  me_col = _sc_mask_gather(nm_pad, idx2).reshape(rc * e, 1)

  hcur = _tc_tile_encode(x, W_atom, b_atom, rc)
  for l in range(nl):
    gathered = _sc_row_gather(hcur, idx2)
    q = _tc_edge(gathered, edge_attr, Wb[l], bb[l], me_col)
    agg = _sc_scatter_add(q, dst, zeros_nd)[:, :n, :]
    hcur = _tc_node_update(hcur, agg.reshape(rc * n, -1), nm_col, eps[l],
                           W1[l], b1[l], W2[l], b2[l])

  y = _tc_final(hcur, nm_col, Wf1, bf1, Wf2, bf2, rc)
  return y.reshape(r, c, -1)
